# baseline (speedup 1.0000x reference)
import functools

import jax
import jax.numpy as jnp
from jax import lax
from jax.experimental import pallas as pl
from jax.experimental.pallas import tpu as pltpu
from jax.experimental.pallas import tpu_sc as plsc

_B, _S, _H, _E, _K = 4, 4096, 4096, 64, 8
_AUX_W = 0.01
_N = _B * _S
_NSPLIT = 2
_NT = _N // _NSPLIT

_MM_BLK = 512


def _mm_body(hs_ref, w_ref, out_ref):
    out_ref[...] = jax.lax.dot_general(
        hs_ref[...], w_ref[...], (((1,), (1,)), ((), ())),
        preferred_element_type=jnp.float32)


_NC, _NS, _L = 2, 16, 16
_NW = _NC * _NS
_CH = 64

_sc_mesh = plsc.VectorSubcoreMesh(
    core_axis_name="c", subcore_axis_name="s", num_cores=_NC, num_subcores=_NS)


def _make_sc_router(n_tok):
    tpw = n_tok // _NW
    nch = tpw // _CH

    @functools.partial(
        pl.kernel,
        out_type=[
            jax.ShapeDtypeStruct((n_tok * 512,), jnp.float32),
            jax.ShapeDtypeStruct((n_tok * _K,), jnp.float32),
            jax.ShapeDtypeStruct((_NW * _E,), jnp.float32),
        ],
        mesh=_sc_mesh,
        compiler_params=pltpu.CompilerParams(needs_layout_passes=False),
        scratch_types=[
            pltpu.VMEM((_CH * _E,), jnp.float32),
            pltpu.VMEM((_CH * _E,), jnp.float32),
            pltpu.VMEM((_CH * 512,), jnp.float32),
            pltpu.VMEM((_CH * 512,), jnp.float32),
            pltpu.VMEM((_CH * _K,), jnp.float32),
            pltpu.VMEM((_CH * _K,), jnp.float32),
            pltpu.VMEM((_CH * 16,), jnp.int32),
            pltpu.VMEM((_CH * 16,), jnp.int32),
            pltpu.VMEM((_E,), jnp.float32),
            pltpu.SemaphoreType.DMA,
            pltpu.SemaphoreType.DMA,
            pltpu.SemaphoreType.DMA,
            pltpu.SemaphoreType.DMA,
            pltpu.SemaphoreType.DMA,
            pltpu.SemaphoreType.DMA,
        ],
    )
    def _sc_router(log_hbm, disp_hbm, comb_hbm, esum_hbm,
                   log_v0, log_v1, disp_v0, disp_v1, comb_v0, comb_v1,
                   idx_v0, idx_v1, acc_v,
                   lin0, lin1, dout0, dout1, cout0, cout1):
        cid = lax.axis_index("c")
        sid = lax.axis_index("s")
        wid = sid * _NC + cid
        base = wid * tpw

        lane = lax.iota(jnp.int32, 16)
        lane_lt8 = lane < _K
        zero16 = jnp.zeros((16,), jnp.float32)
        one16 = jnp.ones((16,), jnp.float32)
        vals_base = [lane, lane + 16, lane + 32, lane + 48]

        log_v = (log_v0, log_v1)
        disp_v = (disp_v0, disp_v1)
        comb_v = (comb_v0, comb_v1)
        idx_v = (idx_v0, idx_v1)
        lin = (lin0, lin1)
        dout = (dout0, dout1)
        cout = (cout0, cout1)

        def _memset(ref, words):
            def mbody(i, carry):
                b = i * 128
                for j in range(8):
                    ref[pl.ds(b + j * 16, 16)] = zero16
                return carry
            lax.fori_loop(0, words // 128, mbody, 0)

        _memset(disp_v0, _CH * 512)
        _memset(disp_v1, _CH * 512)
        for j in range(_E // 16):
            acc_v[pl.ds(j * 16, 16)] = zero16

        def _log_slice(c):
            return log_hbm.at[pl.ds(base * _E + c * (_CH * _E), _CH * _E)]

        def _disp_slice(c):
            return disp_hbm.at[pl.ds((base + c * _CH) * 512, _CH * 512)]

        def _comb_slice(c):
            return comb_hbm.at[pl.ds((base + c * _CH) * _K, _CH * _K)]

        pltpu.async_copy(_log_slice(0), log_v0, lin0)
        pltpu.async_copy(_log_slice(1), log_v1, lin1)

        def _merge(a, b):
            mk = jnp.where(lane_lt8, a[0], lax.rev(b[0], (0,)))
            mv = jnp.where(lane_lt8, a[1], lax.rev(b[1], (0,)))
            return plsc.sort_key_val(mk, mv, descending=True)

        def _chunk(c, s):
            lv, dv, cv, iv = log_v[s], disp_v[s], comb_v[s], idx_v[s]
            pltpu.make_async_copy(_log_slice(c), lv, lin[s]).wait()

            def tok(t, carry):
                off = t * _E
                pairs = [
                    plsc.sort_key_val(lv[pl.ds(off + 16 * j, 16)],
                                      vals_base[j], descending=True)
                    for j in range(4)
                ]
                fk, fv = _merge(_merge(pairs[0], pairs[1]),
                                _merge(pairs[2], pairs[3]))
                kmax = jnp.max(fk)
                ex = jnp.where(lane_lt8, jnp.exp(fk - kmax), 0.0)
                wn = ex / jnp.sum(ex)
                sidx = t * 512 + (lane & 7) * _E + fv
                plsc.store_scatter(dv, [sidx], one16, mask=lane_lt8)
                iv[pl.ds(t * 16, 16)] = sidx
                plsc.store_scatter(cv, [t * _K + lane], wn, mask=lane_lt8)
                plsc.addupdate_scatter(acc_v, [fv], wn, mask=lane_lt8)
                return carry

            lax.fori_loop(0, _CH, tok, 0)
            pltpu.async_copy(dv, _disp_slice(c), dout[s])
            pltpu.async_copy(cv, _comb_slice(c), cout[s])

        def _cleanup(c, s):
            dv, cv, iv = disp_v[s], comb_v[s], idx_v[s]
            pltpu.make_async_copy(dv, _disp_slice(c), dout[s]).wait()
            pltpu.make_async_copy(cv, _comb_slice(c), cout[s]).wait()

            def clb(t, carry):
                sidx = iv[pl.ds(t * 16, 16)]
                plsc.store_scatter(dv, [sidx], zero16, mask=lane_lt8)
                return carry

            lax.fori_loop(0, _CH, clb, 0)

        def mloop(m, carry):
            for s in range(2):
                c = m * 2 + s

                @pl.when(m > 0)
                def _():
                    _cleanup(c - 2, s)

                _chunk(c, s)

                @pl.when(m < nch // 2 - 1)
                def _():
                    pltpu.async_copy(_log_slice(c + 2), log_v[s], lin[s])

            return carry

        lax.fori_loop(0, nch // 2, mloop, 0)

        for s in range(2):
            c = nch - 2 + s
            pltpu.make_async_copy(disp_v[s], _disp_slice(c), dout[s]).wait()
            pltpu.make_async_copy(comb_v[s], _comb_slice(c), cout[s]).wait()

        pltpu.sync_copy(acc_v, esum_hbm.at[pl.ds(wid * _E, _E)])

    return _sc_router


_sc_router_slice = _make_sc_router(_NT)


def _aux_body(es_ref, aux_ref):
    s = jnp.sum(es_ref[...], axis=0)
    aux_ref[0, 0] = jnp.sum(s * s) * (_AUX_W / _N)


def _matmul(hs_slice):
    return pl.pallas_call(
        _mm_body,
        grid=(_NT // _MM_BLK,),
        in_specs=[
            pl.BlockSpec((_MM_BLK, _H), lambda i: (i, 0)),
            pl.BlockSpec((_E, _H), lambda i: (0, 0)),
        ],
        out_specs=pl.BlockSpec((_MM_BLK, _E), lambda i: (i, 0)),
        out_shape=jax.ShapeDtypeStruct((_NT, _E), jnp.float32),
    )


def kernel(hidden_states, W):
    hs = hidden_states.reshape(_N, _H)
    disp_parts, comb_parts, esum_parts = [], [], []
    mm = _matmul(None)
    for i in range(_NSPLIT):
        hs_i = lax.slice_in_dim(hs, i * _NT, (i + 1) * _NT, axis=0)
        logits = mm(hs_i, W)
        d, cb, es = _sc_router_slice(logits.reshape(_NT * _E))
        disp_parts.append(d)
        comb_parts.append(cb)
        esum_parts.append(es)

    esum = jnp.concatenate(esum_parts).reshape(_NSPLIT * _NW, _E)
    aux = pl.pallas_call(
        _aux_body,
        in_specs=[pl.BlockSpec((_NSPLIT * _NW, _E), lambda: (0, 0))],
        out_specs=pl.BlockSpec(memory_space=pltpu.SMEM),
        out_shape=jax.ShapeDtypeStruct((1, 1), jnp.float32),
    )(esum)

    disp = jnp.concatenate(disp_parts)
    comb = jnp.concatenate(comb_parts)
    dispatch_mask = disp.reshape(_N, _K, _E)
    combine_weights = comb.reshape(_B, _S, _K, 1)
    return dispatch_mask, combine_weights, aux[0, 0]


# speedup vs baseline: 3.9958x; 3.9958x over previous
import jax
import jax.numpy as jnp
from jax.experimental import pallas as pl
from jax.experimental.pallas import tpu as pltpu

_B, _S, _H, _E, _K = 4, 4096, 4096, 64, 8
_AUX_W = 0.01
_N = _B * _S
_BLK = 512
_GRID = _N // _BLK


def _fused_body(hs_ref, w_ref, disp_ref, comb_ref, aux_ref, esum_ref):
    i = pl.program_id(0)
    logits = jax.lax.dot_general(
        w_ref[...], hs_ref[...], (((1,), (1,)), ((), ())),
        preferred_element_type=jnp.float32)

    erow = jax.lax.broadcasted_iota(jnp.int32, (_E, _BLK), 0)
    work = logits
    vals, idxs = [], []
    for _ in range(_K):
        m = jnp.max(work, axis=0, keepdims=True)
        amax = jnp.min(jnp.where(work == m, erow, _E), axis=0, keepdims=True)
        vals.append(m)
        idxs.append(amax)
        work = jnp.where(erow == amax, -jnp.inf, work)
    sel_vals = jnp.concatenate(vals, axis=0)
    sel_idx = jnp.concatenate(idxs, axis=0)

    ex = jnp.exp(sel_vals - sel_vals[0:1, :])
    wn = ex / jnp.sum(ex, axis=0, keepdims=True)
    comb_ref[...] = wn

    es_acc = jnp.zeros((_E, _BLK), jnp.float32)
    for k in range(_K):
        eq = (sel_idx[k:k + 1, :] == erow).astype(jnp.float32)
        disp_ref[k, :, :] = eq
        es_acc = es_acc + eq * wn[k:k + 1, :]
    es = jnp.sum(es_acc, axis=1, keepdims=True)

    @pl.when(i == 0)
    def _():
        esum_ref[...] = jnp.zeros_like(esum_ref)

    esum_ref[...] += es

    @pl.when(i == _GRID - 1)
    def _():
        s = esum_ref[:, 0]
        aux_ref[0, 0] = jnp.sum(s * s) * (_AUX_W / _N)


def kernel(hidden_states, W):
    hs = hidden_states.reshape(_N, _H)
    disp, comb, aux = pl.pallas_call(
        _fused_body,
        grid=(_GRID,),
        in_specs=[
            pl.BlockSpec((_BLK, _H), lambda i: (i, 0)),
            pl.BlockSpec((_E, _H), lambda i: (0, 0)),
        ],
        out_specs=[
            pl.BlockSpec((_K, _E, _BLK), lambda i: (0, 0, i)),
            pl.BlockSpec((_K, _BLK), lambda i: (0, i)),
            pl.BlockSpec(memory_space=pltpu.SMEM),
        ],
        out_shape=[
            jax.ShapeDtypeStruct((_K, _E, _N), jnp.float32),
            jax.ShapeDtypeStruct((_K, _N), jnp.float32),
            jax.ShapeDtypeStruct((1, 1), jnp.float32),
        ],
        scratch_shapes=[pltpu.VMEM((_E, 1), jnp.float32)],
    )(hs, W)
    dispatch_mask = jnp.transpose(disp, (2, 0, 1))
    combine_weights = jnp.transpose(comb, (1, 0)).reshape(_B, _S, _K, 1)
    return dispatch_mask, combine_weights, aux[0, 0]


# speedup vs baseline: 4.2817x; 1.0715x over previous
import jax
import jax.numpy as jnp
from jax.experimental import pallas as pl
from jax.experimental.pallas import tpu as pltpu

_B, _S, _H, _E, _K = 4, 4096, 4096, 64, 8
_AUX_W = 0.01
_N = _B * _S
_BLK = 1024
_GRID = _N // _BLK


def _fused_body(hs_ref, w_ref, disp_ref, comb_ref, aux_ref, esum_ref):
    i = pl.program_id(0)
    logits = jax.lax.dot_general(
        w_ref[...], hs_ref[...], (((1,), (1,)), ((), ())),
        preferred_element_type=jnp.float32)

    erow = jax.lax.broadcasted_iota(jnp.int32, (_E, _BLK), 0)
    work = logits
    vals, idxs = [], []
    for _ in range(_K):
        m = jnp.max(work, axis=0, keepdims=True)
        amax = jnp.min(jnp.where(work == m, erow, _E), axis=0, keepdims=True)
        vals.append(m)
        idxs.append(amax)
        work = jnp.where(erow == amax, -jnp.inf, work)
    sel_vals = jnp.concatenate(vals, axis=0)
    sel_idx = jnp.concatenate(idxs, axis=0)

    ex = jnp.exp(sel_vals - sel_vals[0:1, :])
    wn = ex / jnp.sum(ex, axis=0, keepdims=True)
    comb_ref[...] = wn

    es_acc = jnp.zeros((_E, _BLK), jnp.float32)
    for k in range(_K):
        eq = (sel_idx[k:k + 1, :] == erow).astype(jnp.float32)
        disp_ref[k, :, :] = eq
        es_acc = es_acc + eq * wn[k:k + 1, :]
    es = jnp.sum(es_acc, axis=1, keepdims=True)

    @pl.when(i == 0)
    def _():
        esum_ref[...] = jnp.zeros_like(esum_ref)

    esum_ref[...] += es

    @pl.when(i == _GRID - 1)
    def _():
        s = esum_ref[:, 0]
        aux_ref[0, 0] = jnp.sum(s * s) * (_AUX_W / _N)


def kernel(hidden_states, W):
    hs = hidden_states.reshape(_N, _H)
    disp, comb, aux = pl.pallas_call(
        _fused_body,
        grid=(_GRID,),
        in_specs=[
            pl.BlockSpec((_BLK, _H), lambda i: (i, 0)),
            pl.BlockSpec((_E, _H), lambda i: (0, 0)),
        ],
        out_specs=[
            pl.BlockSpec((_K, _E, _BLK), lambda i: (0, 0, i)),
            pl.BlockSpec((_K, _BLK), lambda i: (0, i)),
            pl.BlockSpec(memory_space=pltpu.SMEM),
        ],
        out_shape=[
            jax.ShapeDtypeStruct((_K, _E, _N), jnp.float32),
            jax.ShapeDtypeStruct((_K, _N), jnp.float32),
            jax.ShapeDtypeStruct((1, 1), jnp.float32),
        ],
        scratch_shapes=[pltpu.VMEM((_E, 1), jnp.float32)],
    )(hs, W)
    dispatch_mask = jnp.transpose(disp, (2, 0, 1))
    combine_weights = jnp.transpose(comb, (1, 0)).reshape(_B, _S, _K, 1)
    return dispatch_mask, combine_weights, aux[0, 0]
